# baseline (speedup 1.0000x reference)
import functools
import math

import jax
import jax.numpy as jnp
from jax import lax
from jax.experimental import pallas as pl
from jax.experimental.pallas import tpu as pltpu
from jax.experimental.pallas import tpu_sc as plsc

D_MODEL = 64
SCALE = math.sqrt(D_MODEL)
NUM_CORES = 2
NUM_SUBCORES = 16
NUM_WORKERS = NUM_CORES * NUM_SUBCORES
LANES = 16


def _emb_call(B, CH):
    b_per_w = B // NUM_WORKERS
    n_chunks = b_per_w // CH
    mesh = plsc.VectorSubcoreMesh(core_axis_name="c", subcore_axis_name="s")

    @functools.partial(
        pl.kernel,
        mesh=mesh,
        out_type=jax.ShapeDtypeStruct((B, D_MODEL), jnp.float32),
        scratch_types=[
            pltpu.VMEM((CH,), jnp.int32),
            pltpu.VMEM((CH, D_MODEL), jnp.float32),
            pltpu.SemaphoreType.DMA,
        ],
        compiler_params=pltpu.CompilerParams(use_tc_tiling_on_sc=False),
    )
    def emb_kernel(idx_hbm, table_hbm, out_hbm, idx_v, rows_v, sem):
        wid = lax.axis_index("s") * NUM_CORES + lax.axis_index("c")
        base = wid * b_per_w

        def chunk_body(ci, carry):
            off = base + ci * CH
            pltpu.sync_copy(idx_hbm.at[pl.ds(off, CH)], idx_v)
            pltpu.async_copy(table_hbm.at[idx_v], rows_v, sem).wait()

            def row_body(r, c2):
                for j in range(D_MODEL // LANES):
                    sl = pl.ds(j * LANES, LANES)
                    rows_v[r, sl] = rows_v[r, sl] * SCALE
                return c2

            lax.fori_loop(0, CH, row_body, 0)
            pltpu.sync_copy(rows_v, out_hbm.at[pl.ds(off, CH)])
            return carry

        lax.fori_loop(0, n_chunks, chunk_body, 0)

    return emb_kernel


def kernel(x, table):
    B = x.shape[0] * x.shape[1]
    xf = x.reshape(B).astype(jnp.int32)
    out = _emb_call(B, 800)(xf, table)
    return out.reshape(x.shape + (D_MODEL,))


# speedup vs baseline: 1.1133x; 1.1133x over previous
import functools
import math

import jax
import jax.numpy as jnp
from jax import lax
from jax.experimental import pallas as pl
from jax.experimental.pallas import tpu as pltpu
from jax.experimental.pallas import tpu_sc as plsc

D_MODEL = 64
SCALE = math.sqrt(D_MODEL)
NUM_CORES = 2
NUM_SUBCORES = 16
NUM_WORKERS = NUM_CORES * NUM_SUBCORES
LANES = 16
UNROLL = 8


def _emb_call(B, CH):
    b_per_w = B // NUM_WORKERS
    n_chunks = b_per_w // CH
    assert n_chunks % 2 == 0
    mesh = plsc.VectorSubcoreMesh(core_axis_name="c", subcore_axis_name="s")

    @functools.partial(
        pl.kernel,
        mesh=mesh,
        out_type=jax.ShapeDtypeStruct((B, D_MODEL), jnp.float32),
        scratch_types=[
            pltpu.VMEM((CH,), jnp.int32),
            pltpu.VMEM((CH,), jnp.int32),
            pltpu.VMEM((2, CH, D_MODEL), jnp.float32),
            pltpu.SemaphoreType.DMA,
            pltpu.SemaphoreType.DMA,
            pltpu.SemaphoreType.DMA,
            pltpu.SemaphoreType.DMA,
            pltpu.SemaphoreType.DMA,
            pltpu.SemaphoreType.DMA,
        ],
        compiler_params=pltpu.CompilerParams(use_tc_tiling_on_sc=False),
    )
    def emb_kernel(idx_hbm, table_hbm, out_hbm, idx0, idx1, rows_v,
                   si0, si1, sg0, sg1, so0, so1):
        wid = lax.axis_index("s") * NUM_CORES + lax.axis_index("c")
        base = wid * b_per_w
        idxs = (idx0, idx1)
        isems = (si0, si1)
        gsems = (sg0, sg1)
        osems = (so0, so1)

        def i_desc(ci, b):
            return pltpu.make_async_copy(
                idx_hbm.at[pl.ds(base + ci * CH, CH)], idxs[b], isems[b]
            )

        def g_desc(ci, b):
            return pltpu.make_async_copy(
                table_hbm.at[idxs[b]], rows_v.at[b], gsems[b]
            )

        def o_desc(ci, b):
            return pltpu.make_async_copy(
                rows_v.at[b], out_hbm.at[pl.ds(base + ci * CH, CH)], osems[b]
            )

        def scale(b):
            def body(i, c):
                for u in range(UNROLL):
                    r = i * UNROLL + u
                    for j in range(D_MODEL // LANES):
                        sl = pl.ds(j * LANES, LANES)
                        rows_v[b, r, sl] = rows_v[b, r, sl] * SCALE
                return c

            lax.fori_loop(0, CH // UNROLL, body, 0)

        i_desc(0, 0).start()
        i_desc(0, 0).wait()
        g_desc(0, 0).start()
        i_desc(1, 1).start()

        def outer(g, carry):
            for b in (0, 1):
                ci_s = 2 * g + b
                ci = ci_s
                nb = 1 - b

                @pl.when(ci >= 1)
                def _():
                    o_desc(ci - 1, nb).wait()

                @pl.when(ci + 1 < n_chunks)
                def _():
                    i_desc(ci + 1, nb).wait()
                    g_desc(ci + 1, nb).start()

                g_desc(ci, b).wait()

                @pl.when(ci + 2 < n_chunks)
                def _():
                    i_desc(ci + 2, b).start()

                scale(b)
                o_desc(ci, b).start()
            return carry

        lax.fori_loop(0, n_chunks // 2, outer, 0)
        o_desc(n_chunks - 1, 1).wait()

    return emb_kernel


def kernel(x, table):
    B = x.shape[0] * x.shape[1]
    xf = x.reshape(B).astype(jnp.int32)
    out = _emb_call(B, 800)(xf, table)
    return out.reshape(x.shape + (D_MODEL,))


# speedup vs baseline: 1.1148x; 1.0013x over previous
import functools
import math

import jax
import jax.numpy as jnp
from jax import lax
from jax.experimental import pallas as pl
from jax.experimental.pallas import tpu as pltpu
from jax.experimental.pallas import tpu_sc as plsc

D_MODEL = 64
SCALE = math.sqrt(D_MODEL)
NUM_CORES = 2
NUM_SUBCORES = 16
NUM_WORKERS = NUM_CORES * NUM_SUBCORES
LANES = 16
UNROLL = 8


def _emb_call(B, CH):
    b_per_w = B // NUM_WORKERS
    n_chunks = b_per_w // CH
    assert n_chunks % 2 == 0
    mesh = plsc.VectorSubcoreMesh(core_axis_name="c", subcore_axis_name="s")

    @functools.partial(
        pl.kernel,
        mesh=mesh,
        out_type=jax.ShapeDtypeStruct((B, D_MODEL), jnp.float32),
        scratch_types=[
            pltpu.VMEM((CH,), jnp.int32),
            pltpu.VMEM((CH,), jnp.int32),
            pltpu.VMEM((2, CH, D_MODEL), jnp.float32),
            pltpu.SemaphoreType.DMA,
            pltpu.SemaphoreType.DMA,
            pltpu.SemaphoreType.DMA,
            pltpu.SemaphoreType.DMA,
            pltpu.SemaphoreType.DMA,
            pltpu.SemaphoreType.DMA,
        ],
        compiler_params=pltpu.CompilerParams(use_tc_tiling_on_sc=False),
    )
    def emb_kernel(idx_hbm, table_hbm, out_hbm, idx0, idx1, rows_v,
                   si0, si1, sg0, sg1, so0, so1):
        wid = lax.axis_index("s") * NUM_CORES + lax.axis_index("c")
        base = wid * b_per_w
        idxs = (idx0, idx1)
        isems = (si0, si1)
        gsems = (sg0, sg1)
        osems = (so0, so1)

        def i_desc(ci, b):
            return pltpu.make_async_copy(
                idx_hbm.at[pl.ds(base + ci * CH, CH)], idxs[b], isems[b]
            )

        def g_desc(ci, b):
            return pltpu.make_async_copy(
                table_hbm.at[idxs[b]], rows_v.at[b], gsems[b]
            )

        def o_desc(ci, b):
            return pltpu.make_async_copy(
                rows_v.at[b], out_hbm.at[pl.ds(base + ci * CH, CH)], osems[b]
            )

        def scale(b):
            @plsc.parallel_loop(0, CH, step=1, unroll=UNROLL)
            def body(r):
                for j in range(D_MODEL // LANES):
                    sl = pl.ds(j * LANES, LANES)
                    rows_v[b, r, sl] = rows_v[b, r, sl] * SCALE

        i_desc(0, 0).start()
        i_desc(0, 0).wait()
        g_desc(0, 0).start()
        i_desc(1, 1).start()

        def outer(g, carry):
            for b in (0, 1):
                ci_s = 2 * g + b
                ci = ci_s
                nb = 1 - b

                @pl.when(ci >= 1)
                def _():
                    o_desc(ci - 1, nb).wait()

                @pl.when(ci + 1 < n_chunks)
                def _():
                    i_desc(ci + 1, nb).wait()
                    g_desc(ci + 1, nb).start()

                g_desc(ci, b).wait()

                @pl.when(ci + 2 < n_chunks)
                def _():
                    i_desc(ci + 2, b).start()

                scale(b)
                o_desc(ci, b).start()
            return carry

        lax.fori_loop(0, n_chunks // 2, outer, 0)
        o_desc(n_chunks - 1, 1).wait()

    return emb_kernel


def kernel(x, table):
    B = x.shape[0] * x.shape[1]
    xf = x.reshape(B).astype(jnp.int32)
    out = _emb_call(B, 800)(xf, table)
    return out.reshape(x.shape + (D_MODEL,))
